# baseline (speedup 1.0000x reference)
import jax
import jax.numpy as jnp
from jax import lax
from jax.experimental import pallas as pl
from jax.experimental.pallas import tpu as pltpu
from jax.experimental.pallas import tpu_sc as plsc

B, S, H, V = 128, 512, 768, 30522
NC, NS, L = 2, 16, 16
NW = NC * NS
P = S // NW
NREG = H // L
C = 2
NIT = B // C
NBUF = 4
LOOK = 2


def _body(ids, word, pos, typ, out,
          idx_v, bias_v, typ_v, *rest):
    dests = list(rest[:NBUF])
    sins = list(rest[NBUF:2 * NBUF])
    souts = list(rest[2 * NBUF:3 * NBUF])

    wid = lax.axis_index("s") * NC + lax.axis_index("c")
    base = wid * P

    pltpu.sync_copy(ids.at[wid], idx_v)
    pltpu.sync_copy(pos.at[pl.ds(base, P)], bias_v)
    pltpu.sync_copy(typ.at[0], typ_v)

    @pl.loop(0, P)
    def _bias_row(r):
        for c in range(NREG):
            s = pl.ds(c * L, L)
            bias_v[r, s] = bias_v[r, s] + typ_v[s]

    def gather(k, j):
        pltpu.async_copy(word.at[idx_v.at[0, pl.ds(k * C * P, C * P)]],
                         dests[j], sins[j])

    def wait_gather(k, j):
        pltpu.make_async_copy(word.at[idx_v.at[0, pl.ds(k * C * P, C * P)]],
                              dests[j], sins[j]).wait()

    def scatter(k, j):
        for h in range(C):
            pltpu.async_copy(dests[j].at[pl.ds(h * P, P)],
                             out.at[k * C + h, pl.ds(base, P)], souts[j])

    def wait_scatter(k, j):
        for h in range(C):
            pltpu.make_async_copy(dests[j].at[pl.ds(h * P, P)],
                                  out.at[k * C + h, pl.ds(base, P)],
                                  souts[j]).wait()

    for j in range(LOOK):
        gather(j, j)

    @pl.loop(0, NIT, step=NBUF)
    def _group(g):
        for j in range(NBUF):
            k = g + j
            dst = dests[j]
            wait_gather(k, j)

            for h in range(C):
                @pl.loop(0, P)
                def _row(r):
                    for c in range(NREG):
                        s = pl.ds(c * L, L)
                        dst[h * P + r, s] = dst[h * P + r, s] + bias_v[r, s]

            scatter(k, j)

            j2 = (j + LOOK) % NBUF
            k_next = k + LOOK

            @pl.when(k_next < NIT)
            def _issue():
                @pl.when(k_next >= NBUF)
                def _drain():
                    wait_scatter(k_next - NBUF, j2)
                gather(k_next, j2)

    for j in range(NBUF):
        wait_scatter(NIT - NBUF + j, j)


def kernel(input_ids, word_emb, pos_emb, type_emb):
    mesh = plsc.VectorSubcoreMesh(core_axis_name="c", subcore_axis_name="s")
    f = pl.kernel(
        _body,
        out_type=jax.ShapeDtypeStruct((B, S, H), jnp.float32),
        mesh=mesh,
        scratch_types=[
            pltpu.VMEM((1, B * P), jnp.int32),
            pltpu.VMEM((P, H), jnp.float32),
            pltpu.VMEM((H,), jnp.float32),
        ] + [pltpu.VMEM((C * P, H), jnp.float32) for _ in range(NBUF)]
          + [pltpu.SemaphoreType.DMA for _ in range(2 * NBUF)],
    )
    ids_re = jnp.transpose(
        input_ids.astype(jnp.int32).reshape(B, NW, P), (1, 0, 2)
    ).reshape(NW, 1, B * P)
    return f(ids_re, word_emb, pos_emb, type_emb)


# speedup vs baseline: 1.6118x; 1.6118x over previous
import jax
import jax.numpy as jnp
from jax import lax
from jax.experimental import pallas as pl
from jax.experimental.pallas import tpu as pltpu
from jax.experimental.pallas import tpu_sc as plsc

B, S, H, V = 128, 512, 768, 30522
NC, NS, L = 2, 16, 16
NW = NC * NS
P = S // NW
NREG = H // L
NBUF = 8
LOOK = 4


def _body(ids, word, pos, typ, out, idx_v, bias_v, typ_v, *rest):
    dests = list(rest[:NBUF])
    sins = list(rest[NBUF:2 * NBUF])
    souts = list(rest[2 * NBUF:3 * NBUF])

    wid = lax.axis_index("s") * NC + lax.axis_index("c")
    base = wid * P

    pltpu.sync_copy(ids.at[wid], idx_v)
    pltpu.sync_copy(pos.at[pl.ds(base, P)], bias_v)
    pltpu.sync_copy(typ.at[0], typ_v)

    @pl.loop(0, P)
    def _bias_row(r):
        for c in range(NREG):
            s = pl.ds(c * L, L)
            bias_v[r, s] = bias_v[r, s] + typ_v[s]

    for j in range(LOOK):
        pltpu.async_copy(word.at[idx_v.at[j]], dests[j], sins[j])

    @pl.loop(0, B, step=NBUF)
    def _group(g):
        for j in range(NBUF):
            b = g + j
            dst = dests[j]
            pltpu.make_async_copy(word.at[idx_v.at[b]], dst, sins[j]).wait()

            @pl.loop(0, P)
            def _row(r):
                for c in range(NREG):
                    s = pl.ds(c * L, L)
                    dst[r, s] = dst[r, s] + bias_v[r, s]

            pltpu.async_copy(dst, out.at[b, pl.ds(base, P)], souts[j])

            j2 = (j + LOOK) % NBUF
            b_next = b + LOOK

            @pl.when(b_next < B)
            def _issue():
                @pl.when(b_next >= NBUF)
                def _drain():
                    pltpu.make_async_copy(
                        dests[j2], out.at[b_next - NBUF, pl.ds(base, P)],
                        souts[j2]).wait()
                pltpu.async_copy(word.at[idx_v.at[b_next]], dests[j2],
                                 sins[j2])

    for j in range(NBUF):
        pltpu.make_async_copy(dests[j], out.at[B - NBUF + j, pl.ds(base, P)],
                              souts[j]).wait()


def kernel(input_ids, word_emb, pos_emb, type_emb):
    mesh = plsc.VectorSubcoreMesh(core_axis_name="c", subcore_axis_name="s")
    f = pl.kernel(
        _body,
        out_type=jax.ShapeDtypeStruct((B, S, H), jnp.float32),
        mesh=mesh,
        scratch_types=[
            pltpu.VMEM((B, P), jnp.int32),
            pltpu.VMEM((P, H), jnp.float32),
            pltpu.VMEM((H,), jnp.float32),
        ] + [pltpu.VMEM((P, H), jnp.float32) for _ in range(NBUF)]
          + [pltpu.SemaphoreType.DMA for _ in range(2 * NBUF)],
    )
    ids_re = jnp.transpose(
        input_ids.astype(jnp.int32).reshape(B, NW, P), (1, 0, 2)
    )
    return f(ids_re, word_emb, pos_emb, type_emb)
